# baseline (speedup 1.0000x reference)
import functools

import jax
import jax.numpy as jnp
from jax import lax
from jax.experimental import pallas as pl
from jax.experimental.pallas import tpu as pltpu
from jax.experimental.pallas import tpu_sc as plsc

N = 10000
H = 128
E = 320000
EL = 100000
CW = 16

NC, NS = 2, 16
NW = NC * NS
ROWS = 2560
RPT = ROWS // NS
G = 16
NG = RPT // G
NPT = 624
TRASH = N
EL_PAD = 102400
DPW = EL_PAD // 128 // NW

_MESH = plsc.VectorSubcoreMesh(core_axis_name="c", subcore_axis_name="s")


def _writeout(s_id, shared, out, width):
  @pl.when(s_id < NS - 1)
  def _():
    base = s_id * NPT
    pltpu.sync_copy(shared.at[pl.ds(base, NPT)], out.at[pl.ds(base, NPT)])

  @pl.when(s_id == NS - 1)
  def _():
    last = (NS - 1) * NPT
    pltpu.sync_copy(shared.at[pl.ds(last, N - last)],
                    out.at[pl.ds(last, N - last)])


@functools.partial(
    pl.kernel,
    out_type=(jax.ShapeDtypeStruct((N, H), jnp.float32),
              jax.ShapeDtypeStruct((N, H), jnp.float32)),
    mesh=_MESH,
    scratch_types=[
        pltpu.VMEM((G, 128), jnp.int32),
        pltpu.VMEM((G, 128), jnp.int32),
        pltpu.VMEM((128, H), jnp.float32),
        pltpu.VMEM((128, H), jnp.float32),
        pltpu.VMEM_SHARED((N + 8, H), jnp.float32),
        pltpu.SemaphoreType.DMA,
        pltpu.SemaphoreType.DMA,
        pltpu.SemaphoreType.DMA,
        pltpu.SemaphoreType.DMA,
    ])
def _segsum(src_f, dst_f, table_f, src_r, dst_r, table_r, acc_f, acc_r,
            idx_all, dst_all, rows0, rows1, acc_s, g0, g1, s0, s1):
  c_id = lax.axis_index("c")
  s_id = lax.axis_index("s")
  base = s_id * NPT

  z16 = jnp.zeros((16,), jnp.float32)

  def zfill(r, carry):
    for k in range(H // 16):
      rows0[r, pl.ds(k * 16, 16)] = z16
    return carry
  lax.fori_loop(0, 128, zfill, 0)

  def zloop(j, carry):
    pltpu.sync_copy(rows0, acc_s.at[pl.ds(base + j * 128, 128)])
    return carry
  lax.fori_loop(0, 5, zloop, 0)

  plsc.subcore_barrier()

  def run(src2d, dst2d, table):
    lo = s_id * RPT

    def group(g, carry):
      gb = lo + g * G
      pltpu.sync_copy(src2d.at[pl.ds(gb, G)], idx_all)
      pltpu.sync_copy(dst2d.at[pl.ds(gb, G)], dst_all)
      pltpu.async_copy(table.at[idx_all.at[0]], rows0, g0)

      def body(i, carry2):
        j0 = 2 * i
        pltpu.async_copy(table.at[idx_all.at[j0 + 1]], rows1, g1)
        pltpu.make_async_copy(table.at[idx_all.at[j0]], rows0, g0).wait()
        pltpu.sync_copy(rows0, acc_s.at[dst_all.at[j0]], add=True)

        @pl.when(i < G // 2 - 1)
        def _():
          pltpu.async_copy(table.at[idx_all.at[j0 + 2]], rows0, g0)
        pltpu.make_async_copy(table.at[idx_all.at[j0 + 1]], rows1, g1).wait()
        pltpu.sync_copy(rows1, acc_s.at[dst_all.at[j0 + 1]], add=True)
        return carry2
      lax.fori_loop(0, G // 2, body, 0)
      return carry
    lax.fori_loop(0, NG, group, 0)

  @pl.when(c_id == 0)
  def _():
    run(src_f, dst_f, table_f)

  @pl.when(c_id == 1)
  def _():
    run(src_r, dst_r, table_r)

  plsc.subcore_barrier()

  @pl.when(c_id == 0)
  def _():
    _writeout(s_id, acc_s, acc_f, H)

  @pl.when(c_id == 1)
  def _():
    _writeout(s_id, acc_s, acc_r, H)


@functools.partial(
    pl.kernel,
    out_type=(jax.ShapeDtypeStruct((N, H), jnp.float32),
              jax.ShapeDtypeStruct((N, H), jnp.float32)),
    mesh=_MESH,
    scratch_types=[
        pltpu.VMEM((G, 128), jnp.int32),
        pltpu.VMEM((128, H), jnp.float32),
        pltpu.VMEM_SHARED((N + 8, H), jnp.float32),
        pltpu.SemaphoreType.DMA,
    ])
def _counts(dst_f, dst_r, cnt_f, cnt_r, dst_all, ones_v, cnt_s, s0):
  c_id = lax.axis_index("c")
  s_id = lax.axis_index("s")
  base = s_id * NPT

  z16 = jnp.zeros((16,), jnp.float32)
  o16 = jnp.ones((16,), jnp.float32)

  def zfill(r, carry):
    for k in range(H // 16):
      ones_v[r, pl.ds(k * 16, 16)] = z16
    return carry
  lax.fori_loop(0, 128, zfill, 0)

  def zloop(j, carry):
    pltpu.sync_copy(ones_v, cnt_s.at[pl.ds(base + j * 128, 128)])
    return carry
  lax.fori_loop(0, 5, zloop, 0)

  def ofill(r, carry):
    for k in range(H // 16):
      ones_v[r, pl.ds(k * 16, 16)] = o16
    return carry
  lax.fori_loop(0, 128, ofill, 0)

  plsc.subcore_barrier()

  def run(dst2d):
    lo = s_id * RPT

    def group(g, carry):
      pltpu.sync_copy(dst2d.at[pl.ds(lo + g * G, G)], dst_all)

      def body(j, carry2):
        pltpu.async_copy(ones_v, cnt_s.at[dst_all.at[j]], s0, add=True)
        return carry2
      lax.fori_loop(0, G, body, 0)

      def drain(j, carry2):
        pltpu.make_async_copy(ones_v, cnt_s.at[dst_all.at[0]], s0).wait()
        return carry2
      lax.fori_loop(0, G, drain, 0)
      return carry
    lax.fori_loop(0, NG, group, 0)

  @pl.when(c_id == 0)
  def _():
    run(dst_f)

  @pl.when(c_id == 1)
  def _():
    run(dst_r)

  plsc.subcore_barrier()

  @pl.when(c_id == 0)
  def _():
    _writeout(s_id, cnt_s, cnt_f, H)

  @pl.when(c_id == 1)
  def _():
    _writeout(s_id, cnt_s, cnt_r, H)


@functools.partial(
    pl.kernel,
    out_type=jax.ShapeDtypeStruct((EL_PAD, 16), jnp.float32),
    mesh=_MESH,
    scratch_types=[
        pltpu.VMEM((128,), jnp.int32),
        pltpu.VMEM((128,), jnp.int32),
        pltpu.VMEM((128,), jnp.int32),
        pltpu.VMEM((128,), jnp.int32),
        pltpu.VMEM((128, H), jnp.float32),
        pltpu.VMEM((128, H), jnp.float32),
        pltpu.VMEM((128, H), jnp.float32),
        pltpu.VMEM((128, H), jnp.float32),
        pltpu.VMEM((128, 16), jnp.float32),
        pltpu.SemaphoreType.DMA,
        pltpu.SemaphoreType.DMA,
        pltpu.SemaphoreType.DMA,
        pltpu.SemaphoreType.DMA,
    ])
def _decoder(el0, el1, oa, ob, res,
             ia0, ib0, ia1, ib1, ra0, rb0, ra1, rb1, rv,
             sa0, sb0, sa1, sb1):
  c_id = lax.axis_index("c")
  s_id = lax.axis_index("s")
  w = c_id * NS + s_id
  base = w * DPW

  def stage_fire(i, ia, ib, ra, rb, sa, sb):
    b0 = (base + i) * 128
    pltpu.sync_copy(el0.at[pl.ds(b0, 128)], ia)
    pltpu.sync_copy(el1.at[pl.ds(b0, 128)], ib)
    pltpu.async_copy(oa.at[ia], ra, sa)
    pltpu.async_copy(ob.at[ib], rb, sb)

  def finish(i, ia, ib, ra, rb, sa, sb):
    pltpu.make_async_copy(oa.at[ia], ra, sa).wait()
    pltpu.make_async_copy(ob.at[ib], rb, sb).wait()

    def row(r, carry):
      acc = ra[r, pl.ds(0, 16)] * rb[r, pl.ds(0, 16)]
      for k in range(1, H // 16):
        acc = acc + ra[r, pl.ds(16 * k, 16)] * rb[r, pl.ds(16 * k, 16)]
      rv[r, pl.ds(0, 16)] = acc
      return carry
    lax.fori_loop(0, 128, row, 0)
    pltpu.sync_copy(rv, res.at[pl.ds((base + i) * 128, 128)])

  stage_fire(0, ia0, ib0, ra0, rb0, sa0, sb0)

  def pair(i, carry):
    stage_fire(2 * i + 1, ia1, ib1, ra1, rb1, sa1, sb1)
    finish(2 * i, ia0, ib0, ra0, rb0, sa0, sb0)
    stage_fire(2 * i + 2, ia0, ib0, ra0, rb0, sa0, sb0)
    finish(2 * i + 1, ia1, ib1, ra1, rb1, sa1, sb1)
    return carry

  lax.fori_loop(0, DPW // 2, pair, 0)
  finish(DPW - 1, ia0, ib0, ra0, rb0, sa0, sb0)


def _lane_sum():
  BR = EL_PAD // 128 // 4

  def body(p_ref, o_ref):
    o_ref[...] = jnp.sum(p_ref[...], axis=2)

  return pl.pallas_call(
      body,
      grid=(4,),
      in_specs=[pl.BlockSpec((BR, 128, 16), lambda i: (i, 0, 0))],
      out_specs=pl.BlockSpec((BR, 128), lambda i: (i, 0)),
      out_shape=jax.ShapeDtypeStruct((EL_PAD // 128, 128), jnp.float32),
  )


_lane_sum_call = _lane_sum()


def _tc_layer(relu):
  BN = 1000

  def body(af_ref, cf_ref, xf_ref, wfl_ref, wfr_ref, bf_ref,
           ar_ref, cr_ref, xr_ref, wrl_ref, wrr_ref, br_ref,
           of_ref, or_ref):
    def one(a, c, x, wl, wr, b):
      mean = a / jnp.maximum(c[:, 0:1], 1.0)
      o = (jnp.dot(mean, wl, preferred_element_type=jnp.float32) + b
           + jnp.dot(x, wr, preferred_element_type=jnp.float32))
      return jnp.maximum(o, 0.0) if relu else o

    of_ref[...] = one(af_ref[...], cf_ref[...], xf_ref[...],
                      wfl_ref[...], wfr_ref[...], bf_ref[...])
    or_ref[...] = one(ar_ref[...], cr_ref[...], xr_ref[...],
                      wrl_ref[...], wrr_ref[...], br_ref[...])

  row_spec = pl.BlockSpec((BN, H), lambda i: (i, 0))
  cnt_spec = pl.BlockSpec((BN, H), lambda i: (i, 0))
  w_spec = pl.BlockSpec((H, H), lambda i: (0, 0))
  b_spec = pl.BlockSpec((1, H), lambda i: (0, 0))
  return pl.pallas_call(
      body,
      grid=(N // BN,),
      in_specs=[row_spec, cnt_spec, row_spec, w_spec, w_spec, b_spec,
                row_spec, cnt_spec, row_spec, w_spec, w_spec, b_spec],
      out_specs=[row_spec, row_spec],
      out_shape=[jax.ShapeDtypeStruct((N, H), jnp.float32),
                 jax.ShapeDtypeStruct((N, H), jnp.float32)],
  )


_tc_relu = _tc_layer(True)
_tc_lin = _tc_layer(False)


def kernel(src_node_id, tgt_node_id, edge_index_fwd, edge_index_rev,
           edge_label_index, emb_src, emb_tgt, W1f_l, W1f_r, W1r_l, W1r_r,
           W2f_l, W2f_r, W2r_l, W2r_r, b1f, b1r, b2f, b2r):
  ep = ROWS * 128 - E
  pad_src = jnp.zeros((ep,), jnp.int32)
  pad_dst = jnp.full((ep,), TRASH, jnp.int32)
  src_f = jnp.concatenate([edge_index_fwd[0], pad_src]).reshape(ROWS, 128)
  dst_f = jnp.concatenate([edge_index_fwd[1], pad_dst]).reshape(ROWS, 128)
  src_r = jnp.concatenate([edge_index_rev[0], pad_src]).reshape(ROWS, 128)
  dst_r = jnp.concatenate([edge_index_rev[1], pad_dst]).reshape(ROWS, 128)
  pad_el = jnp.zeros((EL_PAD - EL,), jnp.int32)
  el0 = jnp.concatenate([edge_label_index[0], pad_el])
  el1 = jnp.concatenate([edge_label_index[1], pad_el])

  cnt_f, cnt_r = _counts(dst_f, dst_r)
  acc1f, acc1r = _segsum(src_f, dst_f, emb_src, src_r, dst_r, emb_tgt)

  h_tgt, h_src = _tc_relu(acc1f, cnt_f, emb_tgt, W1f_l, W1f_r,
                          b1f.reshape(1, H),
                          acc1r, cnt_r, emb_src, W1r_l, W1r_r,
                          b1r.reshape(1, H))

  acc2f, acc2r = _segsum(src_f, dst_f, h_src, src_r, dst_r, h_tgt)

  o_tgt, o_src = _tc_lin(acc2f, cnt_f, h_tgt, W2f_l, W2f_r,
                         b2f.reshape(1, H),
                         acc2r, cnt_r, h_src, W2r_l, W2r_r,
                         b2r.reshape(1, H))

  partial = _decoder(el0, el1, o_src, o_tgt)
  out2d = _lane_sum_call(partial.reshape(EL_PAD // 128, 128, 16))
  return out2d.reshape(EL_PAD)[:EL]


# speedup vs baseline: 1.0148x; 1.0148x over previous
import functools

import jax
import jax.numpy as jnp
from jax import lax
from jax.experimental import pallas as pl
from jax.experimental.pallas import tpu as pltpu
from jax.experimental.pallas import tpu_sc as plsc

N = 10000
H = 128
E = 320000
EL = 100000
CW = 16

NC, NS = 2, 16
NW = NC * NS
ROWS = 2560
RPT = ROWS // NS
G = 32
NG = RPT // G
NPT = 624
TRASH = N
EL_PAD = 102400
DPW = EL_PAD // 128 // NW

_MESH = plsc.VectorSubcoreMesh(core_axis_name="c", subcore_axis_name="s")


def _writeout(s_id, shared, out, width):
  @pl.when(s_id < NS - 1)
  def _():
    base = s_id * NPT
    pltpu.sync_copy(shared.at[pl.ds(base, NPT)], out.at[pl.ds(base, NPT)])

  @pl.when(s_id == NS - 1)
  def _():
    last = (NS - 1) * NPT
    pltpu.sync_copy(shared.at[pl.ds(last, N - last)],
                    out.at[pl.ds(last, N - last)])


@functools.partial(
    pl.kernel,
    out_type=(jax.ShapeDtypeStruct((N, H), jnp.float32),
              jax.ShapeDtypeStruct((N, H), jnp.float32)),
    mesh=_MESH,
    scratch_types=[
        pltpu.VMEM((G, 128), jnp.int32),
        pltpu.VMEM((G, 128), jnp.int32),
        pltpu.VMEM((128, H), jnp.float32),
        pltpu.VMEM((128, H), jnp.float32),
        pltpu.VMEM_SHARED((N + 8, H), jnp.float32),
        pltpu.SemaphoreType.DMA,
        pltpu.SemaphoreType.DMA,
        pltpu.SemaphoreType.DMA,
        pltpu.SemaphoreType.DMA,
    ])
def _segsum(src_f, dst_f, table_f, src_r, dst_r, table_r, acc_f, acc_r,
            idx_all, dst_all, rows0, rows1, acc_s, g0, g1, s0, s1):
  c_id = lax.axis_index("c")
  s_id = lax.axis_index("s")
  base = s_id * NPT

  z16 = jnp.zeros((16,), jnp.float32)

  def zfill(r, carry):
    for k in range(H // 16):
      rows0[r, pl.ds(k * 16, 16)] = z16
    return carry
  lax.fori_loop(0, 128, zfill, 0)

  def zloop(j, carry):
    pltpu.sync_copy(rows0, acc_s.at[pl.ds(base + j * 128, 128)])
    return carry
  lax.fori_loop(0, 5, zloop, 0)

  plsc.subcore_barrier()

  def run(src2d, dst2d, table):
    lo = s_id * RPT

    def group(g, carry):
      gb = lo + g * G
      pltpu.sync_copy(src2d.at[pl.ds(gb, G)], idx_all)
      pltpu.sync_copy(dst2d.at[pl.ds(gb, G)], dst_all)
      pltpu.async_copy(table.at[idx_all.at[0]], rows0, g0)

      def body(i, carry2):
        j0 = 2 * i
        pltpu.async_copy(table.at[idx_all.at[j0 + 1]], rows1, g1)
        pltpu.make_async_copy(table.at[idx_all.at[j0]], rows0, g0).wait()
        pltpu.sync_copy(rows0, acc_s.at[dst_all.at[j0]], add=True)

        @pl.when(i < G // 2 - 1)
        def _():
          pltpu.async_copy(table.at[idx_all.at[j0 + 2]], rows0, g0)
        pltpu.make_async_copy(table.at[idx_all.at[j0 + 1]], rows1, g1).wait()
        pltpu.sync_copy(rows1, acc_s.at[dst_all.at[j0 + 1]], add=True)
        return carry2
      lax.fori_loop(0, G // 2, body, 0)
      return carry
    lax.fori_loop(0, NG, group, 0)

  @pl.when(c_id == 0)
  def _():
    run(src_f, dst_f, table_f)

  @pl.when(c_id == 1)
  def _():
    run(src_r, dst_r, table_r)

  plsc.subcore_barrier()

  @pl.when(c_id == 0)
  def _():
    _writeout(s_id, acc_s, acc_f, H)

  @pl.when(c_id == 1)
  def _():
    _writeout(s_id, acc_s, acc_r, H)


@functools.partial(
    pl.kernel,
    out_type=(jax.ShapeDtypeStruct((N, H), jnp.float32),
              jax.ShapeDtypeStruct((N, H), jnp.float32)),
    mesh=_MESH,
    scratch_types=[
        pltpu.VMEM((G, 128), jnp.int32),
        pltpu.VMEM((128, H), jnp.float32),
        pltpu.VMEM_SHARED((N + 8, H), jnp.float32),
        pltpu.SemaphoreType.DMA,
    ])
def _counts(dst_f, dst_r, cnt_f, cnt_r, dst_all, ones_v, cnt_s, s0):
  c_id = lax.axis_index("c")
  s_id = lax.axis_index("s")
  base = s_id * NPT

  z16 = jnp.zeros((16,), jnp.float32)
  o16 = jnp.ones((16,), jnp.float32)

  def zfill(r, carry):
    for k in range(H // 16):
      ones_v[r, pl.ds(k * 16, 16)] = z16
    return carry
  lax.fori_loop(0, 128, zfill, 0)

  def zloop(j, carry):
    pltpu.sync_copy(ones_v, cnt_s.at[pl.ds(base + j * 128, 128)])
    return carry
  lax.fori_loop(0, 5, zloop, 0)

  def ofill(r, carry):
    for k in range(H // 16):
      ones_v[r, pl.ds(k * 16, 16)] = o16
    return carry
  lax.fori_loop(0, 128, ofill, 0)

  plsc.subcore_barrier()

  def run(dst2d):
    lo = s_id * RPT

    def group(g, carry):
      pltpu.sync_copy(dst2d.at[pl.ds(lo + g * G, G)], dst_all)

      def body(j, carry2):
        pltpu.async_copy(ones_v, cnt_s.at[dst_all.at[j]], s0, add=True)
        return carry2
      lax.fori_loop(0, G, body, 0)

      def drain(j, carry2):
        pltpu.make_async_copy(ones_v, cnt_s.at[dst_all.at[0]], s0).wait()
        return carry2
      lax.fori_loop(0, G, drain, 0)
      return carry
    lax.fori_loop(0, NG, group, 0)

  @pl.when(c_id == 0)
  def _():
    run(dst_f)

  @pl.when(c_id == 1)
  def _():
    run(dst_r)

  plsc.subcore_barrier()

  @pl.when(c_id == 0)
  def _():
    _writeout(s_id, cnt_s, cnt_f, H)

  @pl.when(c_id == 1)
  def _():
    _writeout(s_id, cnt_s, cnt_r, H)


@functools.partial(
    pl.kernel,
    out_type=jax.ShapeDtypeStruct((EL_PAD, 16), jnp.float32),
    mesh=_MESH,
    scratch_types=[
        pltpu.VMEM((128,), jnp.int32),
        pltpu.VMEM((128,), jnp.int32),
        pltpu.VMEM((128,), jnp.int32),
        pltpu.VMEM((128,), jnp.int32),
        pltpu.VMEM((128, H), jnp.float32),
        pltpu.VMEM((128, H), jnp.float32),
        pltpu.VMEM((128, H), jnp.float32),
        pltpu.VMEM((128, H), jnp.float32),
        pltpu.VMEM((128, 16), jnp.float32),
        pltpu.SemaphoreType.DMA,
        pltpu.SemaphoreType.DMA,
        pltpu.SemaphoreType.DMA,
        pltpu.SemaphoreType.DMA,
    ])
def _decoder(el0, el1, oa, ob, res,
             ia0, ib0, ia1, ib1, ra0, rb0, ra1, rb1, rv,
             sa0, sb0, sa1, sb1):
  c_id = lax.axis_index("c")
  s_id = lax.axis_index("s")
  w = c_id * NS + s_id
  base = w * DPW

  def stage_fire(i, ia, ib, ra, rb, sa, sb):
    b0 = (base + i) * 128
    pltpu.sync_copy(el0.at[pl.ds(b0, 128)], ia)
    pltpu.sync_copy(el1.at[pl.ds(b0, 128)], ib)
    pltpu.async_copy(oa.at[ia], ra, sa)
    pltpu.async_copy(ob.at[ib], rb, sb)

  def finish(i, ia, ib, ra, rb, sa, sb):
    pltpu.make_async_copy(oa.at[ia], ra, sa).wait()
    pltpu.make_async_copy(ob.at[ib], rb, sb).wait()

    def row(r, carry):
      acc = ra[r, pl.ds(0, 16)] * rb[r, pl.ds(0, 16)]
      for k in range(1, H // 16):
        acc = acc + ra[r, pl.ds(16 * k, 16)] * rb[r, pl.ds(16 * k, 16)]
      rv[r, pl.ds(0, 16)] = acc
      return carry
    lax.fori_loop(0, 128, row, 0)
    pltpu.sync_copy(rv, res.at[pl.ds((base + i) * 128, 128)])

  stage_fire(0, ia0, ib0, ra0, rb0, sa0, sb0)

  def pair(i, carry):
    stage_fire(2 * i + 1, ia1, ib1, ra1, rb1, sa1, sb1)
    finish(2 * i, ia0, ib0, ra0, rb0, sa0, sb0)
    stage_fire(2 * i + 2, ia0, ib0, ra0, rb0, sa0, sb0)
    finish(2 * i + 1, ia1, ib1, ra1, rb1, sa1, sb1)
    return carry

  lax.fori_loop(0, DPW // 2, pair, 0)
  finish(DPW - 1, ia0, ib0, ra0, rb0, sa0, sb0)


def _lane_sum():
  BR = EL_PAD // 128 // 4

  def body(p_ref, o_ref):
    o_ref[...] = jnp.sum(p_ref[...], axis=2)

  return pl.pallas_call(
      body,
      grid=(4,),
      in_specs=[pl.BlockSpec((BR, 128, 16), lambda i: (i, 0, 0))],
      out_specs=pl.BlockSpec((BR, 128), lambda i: (i, 0)),
      out_shape=jax.ShapeDtypeStruct((EL_PAD // 128, 128), jnp.float32),
  )


_lane_sum_call = _lane_sum()


def _tc_layer(relu):
  BN = 1000

  def body(af_ref, cf_ref, xf_ref, wfl_ref, wfr_ref, bf_ref,
           ar_ref, cr_ref, xr_ref, wrl_ref, wrr_ref, br_ref,
           of_ref, or_ref):
    def one(a, c, x, wl, wr, b):
      mean = a / jnp.maximum(c[:, 0:1], 1.0)
      o = (jnp.dot(mean, wl, preferred_element_type=jnp.float32) + b
           + jnp.dot(x, wr, preferred_element_type=jnp.float32))
      return jnp.maximum(o, 0.0) if relu else o

    of_ref[...] = one(af_ref[...], cf_ref[...], xf_ref[...],
                      wfl_ref[...], wfr_ref[...], bf_ref[...])
    or_ref[...] = one(ar_ref[...], cr_ref[...], xr_ref[...],
                      wrl_ref[...], wrr_ref[...], br_ref[...])

  row_spec = pl.BlockSpec((BN, H), lambda i: (i, 0))
  cnt_spec = pl.BlockSpec((BN, H), lambda i: (i, 0))
  w_spec = pl.BlockSpec((H, H), lambda i: (0, 0))
  b_spec = pl.BlockSpec((1, H), lambda i: (0, 0))
  return pl.pallas_call(
      body,
      grid=(N // BN,),
      in_specs=[row_spec, cnt_spec, row_spec, w_spec, w_spec, b_spec,
                row_spec, cnt_spec, row_spec, w_spec, w_spec, b_spec],
      out_specs=[row_spec, row_spec],
      out_shape=[jax.ShapeDtypeStruct((N, H), jnp.float32),
                 jax.ShapeDtypeStruct((N, H), jnp.float32)],
  )


_tc_relu = _tc_layer(True)
_tc_lin = _tc_layer(False)


def kernel(src_node_id, tgt_node_id, edge_index_fwd, edge_index_rev,
           edge_label_index, emb_src, emb_tgt, W1f_l, W1f_r, W1r_l, W1r_r,
           W2f_l, W2f_r, W2r_l, W2r_r, b1f, b1r, b2f, b2r):
  ep = ROWS * 128 - E
  pad_src = jnp.zeros((ep,), jnp.int32)
  pad_dst = jnp.full((ep,), TRASH, jnp.int32)
  src_f = jnp.concatenate([edge_index_fwd[0], pad_src]).reshape(ROWS, 128)
  dst_f = jnp.concatenate([edge_index_fwd[1], pad_dst]).reshape(ROWS, 128)
  src_r = jnp.concatenate([edge_index_rev[0], pad_src]).reshape(ROWS, 128)
  dst_r = jnp.concatenate([edge_index_rev[1], pad_dst]).reshape(ROWS, 128)
  pad_el = jnp.zeros((EL_PAD - EL,), jnp.int32)
  el0 = jnp.concatenate([edge_label_index[0], pad_el])
  el1 = jnp.concatenate([edge_label_index[1], pad_el])

  cnt_f, cnt_r = _counts(dst_f, dst_r)
  acc1f, acc1r = _segsum(src_f, dst_f, emb_src, src_r, dst_r, emb_tgt)

  h_tgt, h_src = _tc_relu(acc1f, cnt_f, emb_tgt, W1f_l, W1f_r,
                          b1f.reshape(1, H),
                          acc1r, cnt_r, emb_src, W1r_l, W1r_r,
                          b1r.reshape(1, H))

  acc2f, acc2r = _segsum(src_f, dst_f, h_src, src_r, dst_r, h_tgt)

  o_tgt, o_src = _tc_lin(acc2f, cnt_f, h_tgt, W2f_l, W2f_r,
                         b2f.reshape(1, H),
                         acc2r, cnt_r, h_src, W2r_l, W2r_r,
                         b2r.reshape(1, H))

  partial = _decoder(el0, el1, o_src, o_tgt)
  out2d = _lane_sum_call(partial.reshape(EL_PAD // 128, 128, 16))
  return out2d.reshape(EL_PAD)[:EL]
